# baseline (speedup 1.0000x reference)
import functools

import jax
import jax.numpy as jnp
import numpy as np
from jax import lax
from jax.experimental import pallas as pl
from jax.experimental.pallas import tpu as pltpu
from jax.experimental.pallas import tpu_sc as plsc

VOCAB = 1000000
D = 64
DP = 128
BATCH = 4096
SEQ = 200
T = BATCH * SEQ

NC, NS = 2, 16
NW = NC * NS
BB = BATCH // NW
W = 2
NSLAB = SEQ // W
SROW = 129


def _pe_table():
    pe = np.array(
        [[pos / np.power(10000, 2 * (j // 2) / D) for j in range(D)]
         if pos != 0 else np.zeros(D) for pos in range(SEQ)])
    pe[1:, 0::2] = np.sin(pe[1:, 0::2])
    pe[1:, 1::2] = np.cos(pe[1:, 1::2])
    return jnp.asarray(pe, dtype=jnp.float32)


_MESH = plsc.VectorSubcoreMesh(
    core_axis_name="c", subcore_axis_name="s", num_cores=NC, num_subcores=NS)


@functools.partial(
    pl.kernel,
    out_type=jax.ShapeDtypeStruct((SEQ, 8, NW, 8, BB), jnp.float32),
    mesh=_MESH,
    scratch_types=[
        pltpu.VMEM((SEQ, BB), jnp.int32),
        pltpu.VMEM((W * BB, D), jnp.float32),
        pltpu.VMEM((W * BB, D), jnp.float32),
        pltpu.VMEM((W * D, SROW), jnp.float32),
        pltpu.VMEM((W * D, SROW), jnp.float32),
        pltpu.VMEM((SEQ, D), jnp.float32),
        pltpu.SemaphoreType.DMA,
        pltpu.SemaphoreType.DMA,
        pltpu.SemaphoreType.DMA,
        pltpu.SemaphoreType.DMA,
    ],
    compiler_params=pltpu.CompilerParams(
        use_tc_tiling_on_sc=False, needs_layout_passes=False),
)
def _embed(x_hbm, table_hbm, pe_hbm, out_hbm,
           idx_v, rows0, rows1, stg0, stg1, pe_v, sg0, sg1, so0, so1):
    wid = lax.axis_index("s") * NC + lax.axis_index("c")
    rows = [rows0, rows1]
    stg = [stg0, stg1]
    sg = [sg0, sg1]
    so = [so0, so1]

    pltpu.sync_copy(pe_hbm, pe_v)
    pltpu.sync_copy(x_hbm.at[wid], idx_v)

    jcol = lax.iota(jnp.int32, 16)
    rowvec = [jcol + 16 * k for k in range(D // 16)]

    def issue_gathers(m, b):
        for k in range(W):
            pltpu.async_copy(
                table_hbm.at[idx_v.at[W * m + k]],
                rows[b].at[pl.ds(k * BB, BB)], sg[b])

    def store_slab(m, b):
        for sp in range(W):
            for jb in range(8):
                pltpu.make_async_copy(
                    stg[b].at[pl.ds((sp * 8 + jb) * 8, 8), pl.ds(0, BB)],
                    out_hbm.at[W * m + sp, jb, wid], so[b]).start()

    def drain_stores(b):
        pltpu.make_async_copy(
            table_hbm.at[pl.ds(0, W * BB)], rows[b], so[b]).wait()

    issue_gathers(0, 0)

    def slab_body(m, b):
        pltpu.make_async_copy(
            table_hbm.at[pl.ds(0, W * BB)], rows[b], sg[b]).wait()

        @pl.when(m + 1 < NSLAB)
        def _():
            issue_gathers(m + 1, 1 - b)

        @pl.when(m >= 2)
        def _():
            drain_stores(b)

        for sp in range(W):
            pe_k = [pe_v[W * m + sp, pl.ds(16 * k, 16)]
                    for k in range(D // 16)]

            @plsc.parallel_loop(0, BB, 1, unroll=4)
            def tok_body(bl):
                colv = jnp.full((16,), bl, jnp.int32)
                r = sp * BB + bl
                for k in range(D // 16):
                    v = rows[b][r, pl.ds(16 * k, 16)] + pe_k[k]
                    plsc.store_scatter(
                        stg[b], [rowvec[k] + sp * D, colv], v)

        store_slab(m, b)

    def pair_body(mm, _):
        for b in range(2):
            slab_body(2 * mm + b, b)
        return ()

    lax.fori_loop(0, NSLAB // 2, pair_body, ())
    drain_stores(0)
    drain_stores(1)


_RB = 512


def _repack_body(t_ref, o_ref):
    o_ref[:, :D] = t_ref[...].T


_repack = pl.pallas_call(
    _repack_body,
    grid=(pl.cdiv(VOCAB, _RB),),
    in_specs=[pl.BlockSpec((D, _RB), lambda i: (0, i))],
    out_specs=pl.BlockSpec((_RB, DP), lambda i: (i, 0)),
    out_shape=jax.ShapeDtypeStruct((VOCAB, DP), jnp.float32),
)


def kernel(x, table):
    table_p = _repack(table.T).reshape(2 * VOCAB, D)
    xt = (x * 2).reshape(NW, BB, SEQ).transpose(0, 2, 1)
    out5 = _embed(xt, table_p, _pe_table())
    return jnp.transpose(out5, (2, 4, 0, 1, 3)).reshape(BATCH, SEQ, D)


# speedup vs baseline: 3.0771x; 3.0771x over previous
import functools

import jax
import jax.numpy as jnp
import numpy as np
from jax import lax
from jax.experimental import pallas as pl
from jax.experimental.pallas import tpu as pltpu
from jax.experimental.pallas import tpu_sc as plsc

VOCAB = 1000000
D = 64
DP = 128
BATCH = 4096
SEQ = 200
T = BATCH * SEQ

NC, NS = 2, 16
NW = NC * NS
BB = BATCH // NW
W = 2
NSLAB = SEQ // W
SROW = 129


def _pe_table():
    pe = np.array(
        [[pos / np.power(10000, 2 * (j // 2) / D) for j in range(D)]
         if pos != 0 else np.zeros(D) for pos in range(SEQ)])
    pe[1:, 0::2] = np.sin(pe[1:, 0::2])
    pe[1:, 1::2] = np.cos(pe[1:, 1::2])
    return jnp.asarray(pe, dtype=jnp.float32)


_MESH = plsc.VectorSubcoreMesh(
    core_axis_name="c", subcore_axis_name="s", num_cores=NC, num_subcores=NS)


@functools.partial(
    pl.kernel,
    out_type=jax.ShapeDtypeStruct((SEQ, 8, NW, 8, BB), jnp.float32),
    mesh=_MESH,
    scratch_types=[
        pltpu.VMEM((SEQ, BB), jnp.int32),
        pltpu.VMEM((W * BB, D), jnp.float32),
        pltpu.VMEM((W * BB, D), jnp.float32),
        pltpu.VMEM((W * D, SROW), jnp.float32),
        pltpu.VMEM((W * D, SROW), jnp.float32),
        pltpu.VMEM((SEQ, D), jnp.float32),
        pltpu.SemaphoreType.DMA,
        pltpu.SemaphoreType.DMA,
        pltpu.SemaphoreType.DMA,
        pltpu.SemaphoreType.DMA,
    ],
    compiler_params=pltpu.CompilerParams(
        use_tc_tiling_on_sc=False, needs_layout_passes=False),
)
def _embed(x_hbm, table_hbm, pe_hbm, out_hbm,
           idx_v, rows0, rows1, stg0, stg1, pe_v, sg0, sg1, so0, so1):
    wid = lax.axis_index("s") * NC + lax.axis_index("c")
    rows = [rows0, rows1]
    stg = [stg0, stg1]
    sg = [sg0, sg1]
    so = [so0, so1]

    pltpu.sync_copy(pe_hbm, pe_v)
    pltpu.sync_copy(x_hbm.at[wid], idx_v)

    jcol = lax.iota(jnp.int32, 16)
    rowvec = [jcol + 16 * k for k in range(D // 16)]

    def issue_gathers(m, b):
        for k in range(W):
            pltpu.async_copy(
                table_hbm.at[idx_v.at[W * m + k]],
                rows[b].at[pl.ds(k * BB, BB)], sg[b])

    def store_slab(m, b):
        for sp in range(W):
            for jb in range(8):
                pltpu.make_async_copy(
                    stg[b].at[pl.ds((sp * 8 + jb) * 8, 8), pl.ds(0, BB)],
                    out_hbm.at[W * m + sp, jb, wid], so[b]).start()

    def drain_stores(b):
        pltpu.make_async_copy(
            table_hbm.at[pl.ds(0, W * BB)], rows[b], so[b]).wait()

    issue_gathers(0, 0)

    def slab_body(m, b):
        pltpu.make_async_copy(
            table_hbm.at[pl.ds(0, W * BB)], rows[b], sg[b]).wait()

        @pl.when(m + 1 < NSLAB)
        def _():
            issue_gathers(m + 1, 1 - b)

        @pl.when(m >= 2)
        def _():
            drain_stores(b)

        for sp in range(W):
            pe_k = [pe_v[W * m + sp, pl.ds(16 * k, 16)]
                    for k in range(D // 16)]

            @plsc.parallel_loop(0, BB, 1, unroll=4)
            def tok_body(bl):
                colv = jnp.full((16,), bl, jnp.int32)
                r = sp * BB + bl
                for k in range(D // 16):
                    v = rows[b][r, pl.ds(16 * k, 16)] + pe_k[k]
                    plsc.store_scatter(
                        stg[b], [rowvec[k] + sp * D, colv], v)

        store_slab(m, b)

    def pair_body(mm, _):
        for b in range(2):
            slab_body(2 * mm + b, b)
        return ()

    lax.fori_loop(0, NSLAB // 2, pair_body, ())
    drain_stores(0)
    drain_stores(1)


_RB = 8192


def _repack_body(t_ref, o_ref):
    o_ref[:, :D] = t_ref[...].T


_repack = pl.pallas_call(
    _repack_body,
    grid=(pl.cdiv(VOCAB, _RB),),
    in_specs=[pl.BlockSpec((D, _RB), lambda i: (0, i))],
    out_specs=pl.BlockSpec((_RB, DP), lambda i: (i, 0)),
    out_shape=jax.ShapeDtypeStruct((VOCAB, DP), jnp.float32),
)


def kernel(x, table):
    table_p = _repack(table.T).reshape(2 * VOCAB, D)
    xt = (x * 2).reshape(NW, BB, SEQ).transpose(0, 2, 1)
    out5 = _embed(xt, table_p, _pe_table())
    return jnp.transpose(out5, (2, 4, 0, 1, 3)).reshape(BATCH, SEQ, D)


# speedup vs baseline: 3.2125x; 1.0440x over previous
import functools

import jax
import jax.numpy as jnp
import numpy as np
from jax import lax
from jax.experimental import pallas as pl
from jax.experimental.pallas import tpu as pltpu
from jax.experimental.pallas import tpu_sc as plsc

VOCAB = 1000000
D = 64
DP = 128
BATCH = 4096
SEQ = 200
T = BATCH * SEQ

NC, NS = 2, 16
NW = NC * NS
BB = BATCH // NW
W = 2
NSLAB = SEQ // W
SROW = 129


def _pe_table():
    pe = np.array(
        [[pos / np.power(10000, 2 * (j // 2) / D) for j in range(D)]
         if pos != 0 else np.zeros(D) for pos in range(SEQ)])
    pe[1:, 0::2] = np.sin(pe[1:, 0::2])
    pe[1:, 1::2] = np.cos(pe[1:, 1::2])
    return jnp.asarray(pe, dtype=jnp.float32)


_MESH = plsc.VectorSubcoreMesh(
    core_axis_name="c", subcore_axis_name="s", num_cores=NC, num_subcores=NS)


@functools.partial(
    pl.kernel,
    out_type=jax.ShapeDtypeStruct((SEQ, 8, NW, 8, BB), jnp.float32),
    mesh=_MESH,
    scratch_types=[
        pltpu.VMEM((SEQ, BB), jnp.int32),
        pltpu.VMEM((W * BB, D), jnp.float32),
        pltpu.VMEM((W * BB, D), jnp.float32),
        pltpu.VMEM((W * D, SROW), jnp.float32),
        pltpu.VMEM((W * D, SROW), jnp.float32),
        pltpu.VMEM((SEQ, D), jnp.float32),
        pltpu.SemaphoreType.DMA,
        pltpu.SemaphoreType.DMA,
        pltpu.SemaphoreType.DMA,
        pltpu.SemaphoreType.DMA,
    ],
    compiler_params=pltpu.CompilerParams(
        use_tc_tiling_on_sc=False, needs_layout_passes=False),
)
def _embed(x_hbm, table_hbm, pe_hbm, out_hbm,
           idx_v, rows0, rows1, stg0, stg1, pe_v, sg0, sg1, so0, so1):
    wid = lax.axis_index("s") * NC + lax.axis_index("c")
    rows = [rows0, rows1]
    stg = [stg0, stg1]
    sg = [sg0, sg1]
    so = [so0, so1]

    pltpu.sync_copy(pe_hbm, pe_v)
    pltpu.sync_copy(x_hbm.at[wid], idx_v)

    jcol = lax.iota(jnp.int32, 16)
    rowvec = [jcol + 16 * k for k in range(D // 16)]

    def issue_gathers(m, b):
        for k in range(W):
            pltpu.async_copy(
                table_hbm.at[idx_v.at[W * m + k]],
                rows[b].at[pl.ds(k * BB, BB)], sg[b])

    def store_slab(m, b):
        for sp in range(W):
            for jb in range(8):
                pltpu.make_async_copy(
                    stg[b].at[pl.ds((sp * 8 + jb) * 8, 8), pl.ds(0, BB)],
                    out_hbm.at[W * m + sp, jb, wid], so[b]).start()

    def drain_stores(b):
        pltpu.make_async_copy(
            table_hbm.at[pl.ds(0, W * BB)], rows[b], so[b]).wait()

    issue_gathers(0, 0)

    def slab_body(m, b):
        pltpu.make_async_copy(
            table_hbm.at[pl.ds(0, W * BB)], rows[b], sg[b]).wait()

        @pl.when(m + 1 < NSLAB)
        def _():
            issue_gathers(m + 1, 1 - b)

        @pl.when(m >= 2)
        def _():
            drain_stores(b)

        for sp in range(W):
            pe_k = [pe_v[W * m + sp, pl.ds(16 * k, 16)]
                    for k in range(D // 16)]

            @plsc.parallel_loop(0, BB, 1, unroll=4)
            def tok_body(bl):
                colv = jnp.full((16,), bl, jnp.int32)
                r = sp * BB + bl
                for k in range(D // 16):
                    v = rows[b][r, pl.ds(16 * k, 16)] + pe_k[k]
                    plsc.store_scatter(
                        stg[b], [rowvec[k] + sp * D, colv], v)

        store_slab(m, b)

    def pair_body(mm, _):
        for b in range(2):
            slab_body(2 * mm + b, b)
        return ()

    lax.fori_loop(0, NSLAB // 2, pair_body, ())
    drain_stores(0)
    drain_stores(1)


_RB = 16384


def _repack_body(t_ref, o_ref):
    o_ref[:, :D] = t_ref[...].T


_repack = pl.pallas_call(
    _repack_body,
    grid=(pl.cdiv(VOCAB, _RB),),
    in_specs=[pl.BlockSpec((D, _RB), lambda i: (0, i))],
    out_specs=pl.BlockSpec((_RB, DP), lambda i: (i, 0)),
    out_shape=jax.ShapeDtypeStruct((VOCAB, DP), jnp.float32),
)


def kernel(x, table):
    table_p = _repack(table.T).reshape(2 * VOCAB, D)
    xt = (x * 2).reshape(NW, BB, SEQ).transpose(0, 2, 1)
    out5 = _embed(xt, table_p, _pe_table())
    return jnp.transpose(out5, (2, 4, 0, 1, 3)).reshape(BATCH, SEQ, D)


# speedup vs baseline: 3.2614x; 1.0152x over previous
import functools

import jax
import jax.numpy as jnp
import numpy as np
from jax import lax
from jax.experimental import pallas as pl
from jax.experimental.pallas import tpu as pltpu
from jax.experimental.pallas import tpu_sc as plsc

VOCAB = 1000000
D = 64
DP = 128
BATCH = 4096
SEQ = 200
T = BATCH * SEQ

NC, NS = 2, 16
NW = NC * NS
BB = BATCH // NW
W = 2
NSLAB = SEQ // W
SROW = 129


def _pe_table():
    pe = np.array(
        [[pos / np.power(10000, 2 * (j // 2) / D) for j in range(D)]
         if pos != 0 else np.zeros(D) for pos in range(SEQ)])
    pe[1:, 0::2] = np.sin(pe[1:, 0::2])
    pe[1:, 1::2] = np.cos(pe[1:, 1::2])
    return jnp.asarray(pe, dtype=jnp.float32)


_MESH = plsc.VectorSubcoreMesh(
    core_axis_name="c", subcore_axis_name="s", num_cores=NC, num_subcores=NS)


@functools.partial(
    pl.kernel,
    out_type=jax.ShapeDtypeStruct((SEQ, 8, NW, 8, BB), jnp.float32),
    mesh=_MESH,
    scratch_types=[
        pltpu.VMEM((SEQ, BB), jnp.int32),
        pltpu.VMEM((W * BB, D), jnp.float32),
        pltpu.VMEM((W * BB, D), jnp.float32),
        pltpu.VMEM((W * D, SROW), jnp.float32),
        pltpu.VMEM((W * D, SROW), jnp.float32),
        pltpu.VMEM((SEQ, D), jnp.float32),
        pltpu.SemaphoreType.DMA,
        pltpu.SemaphoreType.DMA,
        pltpu.SemaphoreType.DMA,
        pltpu.SemaphoreType.DMA,
    ],
    compiler_params=pltpu.CompilerParams(
        use_tc_tiling_on_sc=False, needs_layout_passes=False),
)
def _embed(x_hbm, table_hbm, pe_hbm, out_hbm,
           idx_v, rows0, rows1, stg0, stg1, pe_v, sg0, sg1, so0, so1):
    wid = lax.axis_index("s") * NC + lax.axis_index("c")
    rows = [rows0, rows1]
    stg = [stg0, stg1]
    sg = [sg0, sg1]
    so = [so0, so1]

    pltpu.sync_copy(pe_hbm, pe_v)
    pltpu.sync_copy(x_hbm.at[wid], idx_v)

    jcol = lax.iota(jnp.int32, 16)
    rowvec = [jcol + 16 * k for k in range(D // 16)]

    def issue_gathers(m, b):
        for k in range(W):
            pltpu.async_copy(
                table_hbm.at[idx_v.at[W * m + k]],
                rows[b].at[pl.ds(k * BB, BB)], sg[b])

    def store_slab(m, b):
        for sp in range(W):
            for jb in range(8):
                pltpu.make_async_copy(
                    stg[b].at[pl.ds((sp * 8 + jb) * 8, 8), pl.ds(0, BB)],
                    out_hbm.at[W * m + sp, jb, wid], so[b]).start()

    def drain_stores(b):
        pltpu.make_async_copy(
            table_hbm.at[pl.ds(0, W * BB)], rows[b], so[b]).wait()

    issue_gathers(0, 0)

    def slab_body(m, b):
        pltpu.make_async_copy(
            table_hbm.at[pl.ds(0, W * BB)], rows[b], sg[b]).wait()

        @pl.when(m + 1 < NSLAB)
        def _():
            issue_gathers(m + 1, 1 - b)

        @pl.when(m >= 2)
        def _():
            drain_stores(b)

        for sp in range(W):
            pe_k = [pe_v[W * m + sp, pl.ds(16 * k, 16)]
                    for k in range(D // 16)]

            @plsc.parallel_loop(0, BB, 1, unroll=4)
            def tok_body(bl):
                colv = jnp.full((16,), bl, jnp.int32)
                r = sp * BB + bl
                for k in range(D // 16):
                    v = rows[b][r, pl.ds(16 * k, 16)] + pe_k[k]
                    plsc.store_scatter(
                        stg[b], [rowvec[k] + sp * D, colv], v)

        store_slab(m, b)

    def pair_body(mm, _):
        for b in range(2):
            slab_body(2 * mm + b, b)
        return ()

    lax.fori_loop(0, NSLAB // 2, pair_body, ())
    drain_stores(0)
    drain_stores(1)


_RB = 32768


def _repack_body(t_ref, o_ref):
    o_ref[:, :D] = t_ref[...].T


_repack = pl.pallas_call(
    _repack_body,
    grid=(pl.cdiv(VOCAB, _RB),),
    in_specs=[pl.BlockSpec((D, _RB), lambda i: (0, i))],
    out_specs=pl.BlockSpec((_RB, DP), lambda i: (i, 0)),
    out_shape=jax.ShapeDtypeStruct((VOCAB, DP), jnp.float32),
)


def kernel(x, table):
    table_p = _repack(table.T).reshape(2 * VOCAB, D)
    xt = (x * 2).reshape(NW, BB, SEQ).transpose(0, 2, 1)
    out5 = _embed(xt, table_p, _pe_table())
    return jnp.transpose(out5, (2, 4, 0, 1, 3)).reshape(BATCH, SEQ, D)
